# baseline (speedup 1.0000x reference)
import jax
import jax.numpy as jnp
from jax import lax
from jax.experimental import pallas as pl
from jax.experimental.pallas import tpu as pltpu
from jax.experimental.pallas import tpu_sc as plsc

F32 = jnp.float32
NC = 2
NS = 16
CH = 128
BR = 5120
G_SEG = 64


def _mesh():
    return plsc.VectorSubcoreMesh(core_axis_name="c", subcore_axis_name="s",
                                  num_cores=NC, num_subcores=NS)


def _make_hist(NP, NCH, NHZ):
    ept = NP // NS
    zpt = NHZ // NS
    hchunks = NCH // NC

    def body(dst3, out, dstv, onesv, zb, hist_sh):
        c = lax.axis_index("c")
        s = lax.axis_index("s")
        pltpu.sync_copy(dst3.at[s], dstv)
        for k in range(CH // 16):
            onesv[pl.ds(k * 16, 16)] = jnp.ones((16,), F32)

        def zb_body(i, carry):
            zb[pl.ds(i * 16, 16)] = jnp.zeros((16,), F32)
            return carry

        lax.fori_loop(0, zpt // 16, zb_body, 0)
        pltpu.sync_copy(zb, hist_sh.at[pl.ds(s * zpt, zpt)])
        plsc.subcore_barrier()

        def step(j, carry):
            pltpu.sync_copy(onesv, hist_sh.at[dstv.at[j]], add=True)
            return carry

        lax.fori_loop(c * hchunks, (c + 1) * hchunks, step, 0)
        plsc.subcore_barrier()
        pltpu.sync_copy(hist_sh.at[pl.ds(s * ept, ept)],
                        out.at[c, pl.ds(s * ept, ept)])

    return pl.kernel(
        body,
        out_type=jax.ShapeDtypeStruct((NC, NP), F32),
        mesh=_mesh(),
        scratch_types=[
            pltpu.VMEM((NCH, CH), jnp.int32),
            pltpu.VMEM((CH,), F32),
            pltpu.VMEM((NHZ // NS,), F32),
            pltpu.VMEM_SHARED((NHZ,), F32),
        ],
    )


def _make_conv(NP, HH, NCH, NZP):
    rpt = NP // NS
    zpt = NZP // NS
    nfull = zpt // CH
    rem = zpt - nfull * CH

    def body(y_hbm, src3, dst3, out, srcv, dstv, rows0, rows1, zb, z_sh,
             sem0, sem1):
        c = lax.axis_index("c")
        s = lax.axis_index("s")
        pltpu.sync_copy(src3.at[s], srcv)
        pltpu.sync_copy(dst3.at[s], dstv)
        yc = y_hbm.at[c]

        def zrow(r, carry):
            for k in range(HH // 16):
                zb[r, pl.ds(k * 16, 16)] = jnp.zeros((16,), F32)
            return carry

        lax.fori_loop(0, CH, zrow, 0)
        base = s * zpt
        for t in range(nfull):
            pltpu.sync_copy(zb, z_sh.at[pl.ds(base + t * CH, CH)])
        if rem:
            pltpu.sync_copy(zb.at[pl.ds(0, rem)],
                            z_sh.at[pl.ds(base + nfull * CH, rem)])
        plsc.subcore_barrier()

        pltpu.async_copy(yc.at[srcv.at[0]], rows0, sem0)

        def step(i, carry):
            j = i * 2
            pltpu.async_copy(yc.at[srcv.at[j + 1]], rows1, sem1)
            pltpu.make_async_copy(yc.at[pl.ds(0, CH)], rows0, sem0).wait()
            pltpu.sync_copy(rows0, z_sh.at[dstv.at[j]], add=True)

            @pl.when(j + 2 < NCH)
            def _():
                pltpu.async_copy(yc.at[srcv.at[j + 2]], rows0, sem0)

            pltpu.make_async_copy(yc.at[pl.ds(0, CH)], rows1, sem1).wait()
            pltpu.sync_copy(rows1, z_sh.at[dstv.at[j + 1]], add=True)
            return carry

        lax.fori_loop(0, NCH // 2, step, 0)
        plsc.subcore_barrier()
        pltpu.sync_copy(z_sh.at[pl.ds(s * rpt, rpt)],
                        out.at[c, pl.ds(s * rpt, rpt)])

    return pl.kernel(
        body,
        out_type=jax.ShapeDtypeStruct((NC, NP, HH), F32),
        mesh=_mesh(),
        compiler_params=pltpu.CompilerParams(use_tc_tiling_on_sc=False),
        scratch_types=[
            pltpu.VMEM((NCH, CH), jnp.int32),
            pltpu.VMEM((NCH, CH), jnp.int32),
            pltpu.VMEM((CH, HH), F32),
            pltpu.VMEM((CH, HH), F32),
            pltpu.VMEM((CH, HH), F32),
            pltpu.VMEM_SHARED((NZP, HH), F32),
            pltpu.SemaphoreType.DMA,
            pltpu.SemaphoreType.DMA,
        ],
    )


def _col(blk):
    n = blk.shape[0]
    return jnp.concatenate(
        [jnp.transpose(blk[a:a + 1, :]) for a in range(n)], axis=0)


def _halves(w):
    hh = w.shape[-1] // 2
    return w[:, :hh], w[:, hh:]


def _t0_body(x_ref, w_ref, hist_ref, y_ref, dinv_ref):
    deg = hist_ref[0] + hist_ref[1] + 1.0
    dinv = lax.rsqrt(deg)
    dinv_ref[...] = dinv
    xw = jnp.dot(x_ref[...], w_ref[...], preferred_element_type=F32)
    xw = xw * _col(dinv)
    yl, yr = _halves(xw)
    y_ref[0] = yl
    y_ref[1] = yr


def _t1_body(z_ref, y_ref, dinv_ref, b1_ref, g1_ref, bt1_ref, g2_ref,
             bt2_ref, w2_ref, y2_ref):
    dinv = _col(dinv_ref[...])
    b1l, b1r = _halves(b1_ref[...])
    hl = jnp.maximum(dinv * (z_ref[0] + y_ref[0]) + b1l, 0.0)
    hr = jnp.maximum(dinv * (z_ref[1] + y_ref[1]) + b1r, 0.0)
    n = hl.shape[1] + hr.shape[1]
    for g_r, b_r in ((g1_ref, bt1_ref), (g2_ref, bt2_ref)):
        mu = (jnp.sum(hl, -1, keepdims=True)
              + jnp.sum(hr, -1, keepdims=True)) / n
        dl = hl - mu
        dr = hr - mu
        var = (jnp.sum(dl * dl, -1, keepdims=True)
               + jnp.sum(dr * dr, -1, keepdims=True)) / n
        rstd = lax.rsqrt(var + 1e-5)
        gl, gr = _halves(g_r[...])
        btl, btr = _halves(b_r[...])
        hl = dl * rstd * gl + btl
        hr = dr * rstd * gr + btr
    w2 = w2_ref[...]
    hh = hl.shape[1]
    y2 = (jnp.dot(hl, w2[:hh], preferred_element_type=F32)
          + jnp.dot(hr, w2[hh:], preferred_element_type=F32)) * dinv
    yl, yr = _halves(y2)
    y2_ref[0] = yl
    y2_ref[1] = yr


def _t2_body(z_ref, y_ref, dinv_ref, b_ref, w_ref, o_ref):
    dinv = _col(dinv_ref[...])
    bl, br = _halves(b_ref[...])
    hl = dinv * (z_ref[0] + y_ref[0]) + bl
    hr = dinv * (z_ref[1] + y_ref[1]) + br
    w = w_ref[...]
    hh = hl.shape[1]
    y = (jnp.dot(hl, w[:hh], preferred_element_type=F32)
         + jnp.dot(hr, w[hh:], preferred_element_type=F32)) * dinv
    yl, yr = _halves(y)
    o_ref[0] = yl
    o_ref[1] = yr


def _shift_down(a, k, fill):
    pad = jnp.full((k,) + a.shape[1:], fill, a.dtype)
    return jnp.concatenate([pad, a[:-k]], axis=0)


def _make_t3(O):
    def body(z_ref, y_ref, dinv_ref, b3_ref, lw1_ref, lb1_ref, lw2_ref,
             lb2_ref, batch_ref, out_ref, gmax_s, gsum_s, cnt_s):
        i = pl.program_id(0)

        @pl.when(i == 0)
        def _():
            gmax_s[...] = jnp.full((G_SEG, O), -jnp.inf, F32)
            gsum_s[...] = jnp.zeros((G_SEG, O), F32)
            cnt_s[...] = jnp.zeros((G_SEG, 1), F32)

        dinv = _col(dinv_ref[...])
        b3l, b3r = _halves(b3_ref[...])
        hl = dinv * (z_ref[0] + y_ref[0]) + b3l
        hr = dinv * (z_ref[1] + y_ref[1]) + b3r
        lw1 = lw1_ref[...]
        hh = hl.shape[1]
        h = (jnp.dot(hl, lw1[:hh], preferred_element_type=F32)
             + jnp.dot(hr, lw1[hh:], preferred_element_type=F32)
             + lb1_ref[...])
        h = jnp.dot(h, lw2_ref[...], preferred_element_type=F32) + lb2_ref[...]

        bvec = _col(batch_ref[...])
        seg = lax.broadcasted_iota(jnp.int32, (1, G_SEG), 1)
        mask = (bvec == seg).astype(F32)
        gsum_s[...] += lax.dot_general(mask, h, (((0,), (0,)), ((), ())),
                                       preferred_element_type=F32)
        ones_col = jnp.ones((h.shape[0], 1), F32)
        bcnt = lax.dot_general(mask, ones_col, (((0,), (0,)), ((), ())),
                               preferred_element_type=F32)
        cnt_s[...] += bcnt

        m = h
        k = 1
        while k < h.shape[0]:
            msh = _shift_down(m, k, -jnp.inf)
            bsh = _shift_down(bvec, k, -1)
            m = jnp.maximum(m, jnp.where(bvec == bsh, msh, -jnp.inf))
            k *= 2
        bnext = jnp.concatenate(
            [bvec[1:], jnp.full((1, 1), G_SEG + 1, bvec.dtype)], axis=0)
        mlast = jnp.where(bvec != bnext, m, 0.0)
        bmax = lax.dot_general(mask, mlast, (((0,), (0,)), ((), ())),
                               preferred_element_type=F32)
        bmax = jnp.where(bcnt > 0.0, bmax, -jnp.inf)
        gmax_s[...] = jnp.maximum(gmax_s[...], bmax)

        @pl.when(i == pl.num_programs(0) - 1)
        def _():
            cnt = jnp.maximum(cnt_s[...], 1.0)
            gmean = gsum_s[...] / cnt
            row = jnp.concatenate([gmax_s[...], gmean], axis=1)
            mx = jnp.max(row, axis=1, keepdims=True)
            ex = jnp.exp(row - mx)
            out_ref[...] = row - mx - jnp.log(
                jnp.sum(ex, axis=1, keepdims=True))

    return body


def _row_spec(shape):
    if len(shape) == 2:
        return pl.BlockSpec((BR, shape[1]), lambda i: (i, 0))
    return pl.BlockSpec((shape[0], BR, shape[2]), lambda i: (0, i, 0))


def _packed_spec(nd):
    if nd == 2:
        return pl.BlockSpec((BR // 128, 128), lambda i: (i, 0))
    return pl.BlockSpec((NC, BR // 128, 128), lambda i: (0, i, 0))


def _full_spec(shape):
    nd = len(shape)
    return pl.BlockSpec(shape, lambda i, _nd=nd: (0,) * _nd)


def kernel(x, edge_index, batch, W1, b1, g1, bt1, g2, bt2, W2, b2, W3, b3,
           LW1, Lb1, LW2, Lb2):
    N, D = x.shape
    E = edge_index.shape[1]
    H = W1.shape[1]
    HH = H // 2
    O = LW2.shape[1]

    NP = ((N + BR - 1) // BR) * BR
    ept0 = -(-E // NS)
    NCH = -(-ept0 // CH)
    if NCH % 2:
        NCH += 1
    ept = NCH * CH
    NZP = ((NP + 16 + NS - 1) // NS) * NS
    NHZ = ((NP + 16 + 16 * NS - 1) // (16 * NS)) * (16 * NS)

    xp = jnp.pad(x, ((0, NP - N), (0, 0)))
    batch_p = jnp.pad(batch, (0, NP - N), constant_values=G_SEG)
    src = edge_index[0]
    dst = edge_index[1]
    idt = dst.dtype
    src2 = jnp.pad(src, (0, NS * ept0 - E)).reshape(NS, ept0)
    src3 = jnp.pad(src2, ((0, 0), (0, ept - ept0))).reshape(NS, NCH, CH)
    dst2 = jnp.concatenate(
        [dst, jnp.full((NS * ept0 - E,), NP, idt)]).reshape(NS, ept0)
    dpad = jnp.broadcast_to(NP + (jnp.arange(ept - ept0, dtype=idt) % 16),
                            (NS, ept - ept0))
    dst3 = jnp.concatenate([dst2, dpad], axis=1).reshape(NS, NCH, CH)

    hist_call = _make_hist(NP, NCH, NHZ)
    conv_call = _make_conv(NP, HH, NCH, NZP)
    nb = NP // BR

    hist = hist_call(dst3)
    hist3 = hist.reshape(NC, NP // 128, 128)

    y1, dinv = pl.pallas_call(
        _t0_body,
        grid=(nb,),
        in_specs=[_row_spec((NP, D)), _full_spec((D, H)), _packed_spec(3)],
        out_specs=[_row_spec((NC, NP, HH)), _packed_spec(2)],
        out_shape=[jax.ShapeDtypeStruct((NC, NP, HH), F32),
                   jax.ShapeDtypeStruct((NP // 128, 128), F32)],
    )(xp, W1, hist3)

    z1 = conv_call(y1, src3, dst3)

    vec = lambda v: v.reshape(1, -1)
    y2 = pl.pallas_call(
        _t1_body,
        grid=(nb,),
        in_specs=[_row_spec((NC, NP, HH)), _row_spec((NC, NP, HH)),
                  _packed_spec(2),
                  _full_spec((1, H)), _full_spec((1, H)), _full_spec((1, H)),
                  _full_spec((1, H)), _full_spec((1, H)),
                  _full_spec((H, H))],
        out_specs=_row_spec((NC, NP, HH)),
        out_shape=jax.ShapeDtypeStruct((NC, NP, HH), F32),
    )(z1, y1, dinv, vec(b1), vec(g1), vec(bt1), vec(g2), vec(bt2), W2)

    z2 = conv_call(y2, src3, dst3)

    y3 = pl.pallas_call(
        _t2_body,
        grid=(nb,),
        in_specs=[_row_spec((NC, NP, HH)), _row_spec((NC, NP, HH)),
                  _packed_spec(2), _full_spec((1, H)),
                  _full_spec((H, H))],
        out_specs=_row_spec((NC, NP, HH)),
        out_shape=jax.ShapeDtypeStruct((NC, NP, HH), F32),
    )(z2, y2, dinv, vec(b2), W3)

    z3 = conv_call(y3, src3, dst3)

    out = pl.pallas_call(
        _make_t3(O),
        grid=(nb,),
        in_specs=[_row_spec((NC, NP, HH)), _row_spec((NC, NP, HH)),
                  _packed_spec(2), _full_spec((1, H)),
                  _full_spec((H, H)), _full_spec((1, H)),
                  _full_spec((H, O)), _full_spec((1, O)),
                  _packed_spec(2)],
        out_specs=pl.BlockSpec((G_SEG, 2 * O), lambda i: (0, 0)),
        out_shape=jax.ShapeDtypeStruct((G_SEG, 2 * O), F32),
        scratch_shapes=[pltpu.VMEM((G_SEG, O), F32),
                        pltpu.VMEM((G_SEG, O), F32),
                        pltpu.VMEM((G_SEG, 1), F32)],
    )(z3, y3, dinv, vec(b3), LW1, vec(Lb1), LW2, vec(Lb2),
      batch_p.astype(jnp.int32).reshape(NP // 128, 128))

    return out


# speedup vs baseline: 1.0108x; 1.0108x over previous
import jax
import jax.numpy as jnp
from jax import lax
from jax.experimental import pallas as pl
from jax.experimental.pallas import tpu as pltpu
from jax.experimental.pallas import tpu_sc as plsc

F32 = jnp.float32
NC = 2
NS = 16
CH = 128
BR = 2048
G_SEG = 64


def _mesh():
    return plsc.VectorSubcoreMesh(core_axis_name="c", subcore_axis_name="s",
                                  num_cores=NC, num_subcores=NS)


def _make_hist(NP, NCH, NHZ):
    ept = NP // NS
    zpt = NHZ // NS
    hchunks = NCH // NC

    def body(dst3, out, dstv, onesv, zb, hist_sh):
        c = lax.axis_index("c")
        s = lax.axis_index("s")
        pltpu.sync_copy(dst3.at[s], dstv)
        for k in range(CH // 16):
            onesv[pl.ds(k * 16, 16)] = jnp.ones((16,), F32)

        def zb_body(i, carry):
            zb[pl.ds(i * 16, 16)] = jnp.zeros((16,), F32)
            return carry

        lax.fori_loop(0, zpt // 16, zb_body, 0)
        pltpu.sync_copy(zb, hist_sh.at[pl.ds(s * zpt, zpt)])
        plsc.subcore_barrier()

        def step(j, carry):
            pltpu.sync_copy(onesv, hist_sh.at[dstv.at[j]], add=True)
            return carry

        lax.fori_loop(c * hchunks, (c + 1) * hchunks, step, 0)
        plsc.subcore_barrier()
        pltpu.sync_copy(hist_sh.at[pl.ds(s * ept, ept)],
                        out.at[c, pl.ds(s * ept, ept)])

    return pl.kernel(
        body,
        out_type=jax.ShapeDtypeStruct((NC, NP), F32),
        mesh=_mesh(),
        scratch_types=[
            pltpu.VMEM((NCH, CH), jnp.int32),
            pltpu.VMEM((CH,), F32),
            pltpu.VMEM((NHZ // NS,), F32),
            pltpu.VMEM_SHARED((NHZ,), F32),
        ],
    )


def _make_conv(NP, HH, NCH, NZP):
    rpt = NP // NS
    zpt = NZP // NS
    nfull = zpt // CH
    rem = zpt - nfull * CH

    def body(y_hbm, src3, dst3, out, srcv, dstv, rows0, rows1, zb, z_sh,
             sem0, sem1):
        c = lax.axis_index("c")
        s = lax.axis_index("s")
        pltpu.sync_copy(src3.at[s], srcv)
        pltpu.sync_copy(dst3.at[s], dstv)
        yc = y_hbm.at[c]

        def zrow(r, carry):
            for k in range(HH // 16):
                zb[r, pl.ds(k * 16, 16)] = jnp.zeros((16,), F32)
            return carry

        lax.fori_loop(0, CH, zrow, 0)
        base = s * zpt
        for t in range(nfull):
            pltpu.sync_copy(zb, z_sh.at[pl.ds(base + t * CH, CH)])
        if rem:
            pltpu.sync_copy(zb.at[pl.ds(0, rem)],
                            z_sh.at[pl.ds(base + nfull * CH, rem)])
        plsc.subcore_barrier()

        pltpu.async_copy(yc.at[srcv.at[0]], rows0, sem0)

        def step(i, carry):
            j = i * 2
            pltpu.async_copy(yc.at[srcv.at[j + 1]], rows1, sem1)
            pltpu.make_async_copy(yc.at[pl.ds(0, CH)], rows0, sem0).wait()
            pltpu.sync_copy(rows0, z_sh.at[dstv.at[j]], add=True)

            @pl.when(j + 2 < NCH)
            def _():
                pltpu.async_copy(yc.at[srcv.at[j + 2]], rows0, sem0)

            pltpu.make_async_copy(yc.at[pl.ds(0, CH)], rows1, sem1).wait()
            pltpu.sync_copy(rows1, z_sh.at[dstv.at[j + 1]], add=True)
            return carry

        lax.fori_loop(0, NCH // 2, step, 0)
        plsc.subcore_barrier()
        pltpu.sync_copy(z_sh.at[pl.ds(s * rpt, rpt)],
                        out.at[c, pl.ds(s * rpt, rpt)])

    return pl.kernel(
        body,
        out_type=jax.ShapeDtypeStruct((NC, NP, HH), F32),
        mesh=_mesh(),
        compiler_params=pltpu.CompilerParams(use_tc_tiling_on_sc=False),
        scratch_types=[
            pltpu.VMEM((NCH, CH), jnp.int32),
            pltpu.VMEM((NCH, CH), jnp.int32),
            pltpu.VMEM((CH, HH), F32),
            pltpu.VMEM((CH, HH), F32),
            pltpu.VMEM((CH, HH), F32),
            pltpu.VMEM_SHARED((NZP, HH), F32),
            pltpu.SemaphoreType.DMA,
            pltpu.SemaphoreType.DMA,
        ],
    )


def _col(blk):
    n = blk.shape[0]
    return jnp.concatenate(
        [jnp.transpose(blk[a:a + 1, :]) for a in range(n)], axis=0)


def _halves(w):
    hh = w.shape[-1] // 2
    return w[:, :hh], w[:, hh:]


def _t0_body(x_ref, w_ref, hist_ref, y_ref, dinv_ref):
    deg = hist_ref[0] + hist_ref[1] + 1.0
    dinv = lax.rsqrt(deg)
    dinv_ref[...] = dinv
    xw = jnp.dot(x_ref[...], w_ref[...], preferred_element_type=F32)
    xw = xw * _col(dinv)
    yl, yr = _halves(xw)
    y_ref[0] = yl
    y_ref[1] = yr


def _t1_body(z_ref, y_ref, dinv_ref, b1_ref, g1_ref, bt1_ref, g2_ref,
             bt2_ref, w2_ref, y2_ref):
    dinv = _col(dinv_ref[...])
    b1l, b1r = _halves(b1_ref[...])
    hl = jnp.maximum(dinv * (z_ref[0] + y_ref[0]) + b1l, 0.0)
    hr = jnp.maximum(dinv * (z_ref[1] + y_ref[1]) + b1r, 0.0)
    n = hl.shape[1] + hr.shape[1]
    for g_r, b_r in ((g1_ref, bt1_ref), (g2_ref, bt2_ref)):
        mu = (jnp.sum(hl, -1, keepdims=True)
              + jnp.sum(hr, -1, keepdims=True)) / n
        dl = hl - mu
        dr = hr - mu
        var = (jnp.sum(dl * dl, -1, keepdims=True)
               + jnp.sum(dr * dr, -1, keepdims=True)) / n
        rstd = lax.rsqrt(var + 1e-5)
        gl, gr = _halves(g_r[...])
        btl, btr = _halves(b_r[...])
        hl = dl * rstd * gl + btl
        hr = dr * rstd * gr + btr
    w2 = w2_ref[...]
    hh = hl.shape[1]
    y2 = (jnp.dot(hl, w2[:hh], preferred_element_type=F32)
          + jnp.dot(hr, w2[hh:], preferred_element_type=F32)) * dinv
    yl, yr = _halves(y2)
    y2_ref[0] = yl
    y2_ref[1] = yr


def _t2_body(z_ref, y_ref, dinv_ref, b_ref, w_ref, o_ref):
    dinv = _col(dinv_ref[...])
    bl, br = _halves(b_ref[...])
    hl = dinv * (z_ref[0] + y_ref[0]) + bl
    hr = dinv * (z_ref[1] + y_ref[1]) + br
    w = w_ref[...]
    hh = hl.shape[1]
    y = (jnp.dot(hl, w[:hh], preferred_element_type=F32)
         + jnp.dot(hr, w[hh:], preferred_element_type=F32)) * dinv
    yl, yr = _halves(y)
    o_ref[0] = yl
    o_ref[1] = yr


def _shift_down(a, k, fill):
    pad = jnp.full((k,) + a.shape[1:], fill, a.dtype)
    return jnp.concatenate([pad, a[:-k]], axis=0)


def _make_t3(O):
    def body(z_ref, y_ref, dinv_ref, b3_ref, lw1_ref, lb1_ref, lw2_ref,
             lb2_ref, batch_ref, out_ref, gmax_s, gsum_s, cnt_s):
        i = pl.program_id(0)

        @pl.when(i == 0)
        def _():
            gmax_s[...] = jnp.full((G_SEG, O), -jnp.inf, F32)
            gsum_s[...] = jnp.zeros((G_SEG, O), F32)
            cnt_s[...] = jnp.zeros((G_SEG, 1), F32)

        dinv = _col(dinv_ref[...])
        b3l, b3r = _halves(b3_ref[...])
        hl = dinv * (z_ref[0] + y_ref[0]) + b3l
        hr = dinv * (z_ref[1] + y_ref[1]) + b3r
        lw1 = lw1_ref[...]
        hh = hl.shape[1]
        h = (jnp.dot(hl, lw1[:hh], preferred_element_type=F32)
             + jnp.dot(hr, lw1[hh:], preferred_element_type=F32)
             + lb1_ref[...])
        h = jnp.dot(h, lw2_ref[...], preferred_element_type=F32) + lb2_ref[...]

        bvec = _col(batch_ref[...])
        seg = lax.broadcasted_iota(jnp.int32, (1, G_SEG), 1)
        mask = (bvec == seg).astype(F32)
        gsum_s[...] += lax.dot_general(mask, h, (((0,), (0,)), ((), ())),
                                       preferred_element_type=F32)
        ones_col = jnp.ones((h.shape[0], 1), F32)
        bcnt = lax.dot_general(mask, ones_col, (((0,), (0,)), ((), ())),
                               preferred_element_type=F32)
        cnt_s[...] += bcnt

        m = h
        k = 1
        while k < h.shape[0]:
            msh = _shift_down(m, k, -jnp.inf)
            bsh = _shift_down(bvec, k, -1)
            m = jnp.maximum(m, jnp.where(bvec == bsh, msh, -jnp.inf))
            k *= 2
        bnext = jnp.concatenate(
            [bvec[1:], jnp.full((1, 1), G_SEG + 1, bvec.dtype)], axis=0)
        mlast = jnp.where(bvec != bnext, m, 0.0)
        bmax = lax.dot_general(mask, mlast, (((0,), (0,)), ((), ())),
                               preferred_element_type=F32)
        bmax = jnp.where(bcnt > 0.0, bmax, -jnp.inf)
        gmax_s[...] = jnp.maximum(gmax_s[...], bmax)

        @pl.when(i == pl.num_programs(0) - 1)
        def _():
            cnt = jnp.maximum(cnt_s[...], 1.0)
            gmean = gsum_s[...] / cnt
            row = jnp.concatenate([gmax_s[...], gmean], axis=1)
            mx = jnp.max(row, axis=1, keepdims=True)
            ex = jnp.exp(row - mx)
            out_ref[...] = row - mx - jnp.log(
                jnp.sum(ex, axis=1, keepdims=True))

    return body


def _row_spec(shape):
    if len(shape) == 2:
        return pl.BlockSpec((BR, shape[1]), lambda i: (i, 0))
    return pl.BlockSpec((shape[0], BR, shape[2]), lambda i: (0, i, 0))


def _packed_spec(nd):
    if nd == 2:
        return pl.BlockSpec((BR // 128, 128), lambda i: (i, 0))
    return pl.BlockSpec((NC, BR // 128, 128), lambda i: (0, i, 0))


def _full_spec(shape):
    nd = len(shape)
    return pl.BlockSpec(shape, lambda i, _nd=nd: (0,) * _nd)


def kernel(x, edge_index, batch, W1, b1, g1, bt1, g2, bt2, W2, b2, W3, b3,
           LW1, Lb1, LW2, Lb2):
    N, D = x.shape
    E = edge_index.shape[1]
    H = W1.shape[1]
    HH = H // 2
    O = LW2.shape[1]

    NP = ((N + BR - 1) // BR) * BR
    ept0 = -(-E // NS)
    NCH = -(-ept0 // CH)
    if NCH % 2:
        NCH += 1
    ept = NCH * CH
    NZP = ((NP + 16 + NS - 1) // NS) * NS
    NHZ = ((NP + 16 + 16 * NS - 1) // (16 * NS)) * (16 * NS)

    xp = jnp.pad(x, ((0, NP - N), (0, 0)))
    batch_p = jnp.pad(batch, (0, NP - N), constant_values=G_SEG)
    src = edge_index[0]
    dst = edge_index[1]
    idt = dst.dtype
    src2 = jnp.pad(src, (0, NS * ept0 - E)).reshape(NS, ept0)
    src3 = jnp.pad(src2, ((0, 0), (0, ept - ept0))).reshape(NS, NCH, CH)
    dst2 = jnp.concatenate(
        [dst, jnp.full((NS * ept0 - E,), NP, idt)]).reshape(NS, ept0)
    dpad = jnp.broadcast_to(NP + (jnp.arange(ept - ept0, dtype=idt) % 16),
                            (NS, ept - ept0))
    dst3 = jnp.concatenate([dst2, dpad], axis=1).reshape(NS, NCH, CH)

    hist_call = _make_hist(NP, NCH, NHZ)
    conv_call = _make_conv(NP, HH, NCH, NZP)
    nb = NP // BR

    hist = hist_call(dst3)
    hist3 = hist.reshape(NC, NP // 128, 128)

    y1, dinv = pl.pallas_call(
        _t0_body,
        grid=(nb,),
        in_specs=[_row_spec((NP, D)), _full_spec((D, H)), _packed_spec(3)],
        out_specs=[_row_spec((NC, NP, HH)), _packed_spec(2)],
        out_shape=[jax.ShapeDtypeStruct((NC, NP, HH), F32),
                   jax.ShapeDtypeStruct((NP // 128, 128), F32)],
    )(xp, W1, hist3)

    z1 = conv_call(y1, src3, dst3)

    vec = lambda v: v.reshape(1, -1)
    y2 = pl.pallas_call(
        _t1_body,
        grid=(nb,),
        in_specs=[_row_spec((NC, NP, HH)), _row_spec((NC, NP, HH)),
                  _packed_spec(2),
                  _full_spec((1, H)), _full_spec((1, H)), _full_spec((1, H)),
                  _full_spec((1, H)), _full_spec((1, H)),
                  _full_spec((H, H))],
        out_specs=_row_spec((NC, NP, HH)),
        out_shape=jax.ShapeDtypeStruct((NC, NP, HH), F32),
    )(z1, y1, dinv, vec(b1), vec(g1), vec(bt1), vec(g2), vec(bt2), W2)

    z2 = conv_call(y2, src3, dst3)

    y3 = pl.pallas_call(
        _t2_body,
        grid=(nb,),
        in_specs=[_row_spec((NC, NP, HH)), _row_spec((NC, NP, HH)),
                  _packed_spec(2), _full_spec((1, H)),
                  _full_spec((H, H))],
        out_specs=_row_spec((NC, NP, HH)),
        out_shape=jax.ShapeDtypeStruct((NC, NP, HH), F32),
    )(z2, y2, dinv, vec(b2), W3)

    z3 = conv_call(y3, src3, dst3)

    out = pl.pallas_call(
        _make_t3(O),
        grid=(nb,),
        in_specs=[_row_spec((NC, NP, HH)), _row_spec((NC, NP, HH)),
                  _packed_spec(2), _full_spec((1, H)),
                  _full_spec((H, H)), _full_spec((1, H)),
                  _full_spec((H, O)), _full_spec((1, O)),
                  _packed_spec(2)],
        out_specs=pl.BlockSpec((G_SEG, 2 * O), lambda i: (0, 0)),
        out_shape=jax.ShapeDtypeStruct((G_SEG, 2 * O), F32),
        scratch_shapes=[pltpu.VMEM((G_SEG, O), F32),
                        pltpu.VMEM((G_SEG, O), F32),
                        pltpu.VMEM((G_SEG, 1), F32)],
    )(z3, y3, dinv, vec(b3), LW1, vec(Lb1), LW2, vec(Lb2),
      batch_p.astype(jnp.int32).reshape(NP // 128, 128))

    return out
